# Initial kernel scaffold; baseline (speedup 1.0000x reference)
import jax
import jax.numpy as jnp
from jax.experimental import pallas as pl


def kernel(x, Wg, bg, Wr, br, Wc1, bc1, Wc2, bc2, W1, b1, W2, b2):
    raise NotImplementedError("write your pallas kernel here")


# speedup vs baseline: 2.0844x; 2.0844x over previous
import functools

import jax
import jax.numpy as jnp
from jax import lax
from jax.experimental import pallas as pl
from jax.experimental.pallas import tpu as pltpu

N = 2048
D = 1024
E = 8
G = 4
PER = 2
H = 2048
BLK = 256
P = 6144
NB = P // BLK
RBLK = 256


def _gelu_exact(v):
    return 0.5 * v * (1.0 + lax.erf(v * 0.7071067811865476))


def _router_body(x_ref, wg_ref, bgr_ref, wr_ref, brr_ref, wc1_ref, bc1_ref,
                 wc2_ref, bc2_ref, i1_ref, i2_ref, p1_ref, p2_ref):
    xb = x_ref[...]
    col = lax.broadcasted_iota(jnp.int32, (RBLK, 128), 1)
    gmask = col < G
    emask = col < E

    glog = jnp.dot(xb, wg_ref[...].T, preferred_element_type=jnp.float32)
    glog = glog + bgr_ref[...]
    glogm = jnp.where(gmask, glog, -jnp.inf)
    gmax = jnp.max(glogm, axis=1, keepdims=True)
    gexp = jnp.where(gmask, jnp.exp(glog - gmax), 0.0)
    gprob = gexp / jnp.sum(gexp, axis=1, keepdims=True)

    m1 = jnp.max(jnp.where(gmask, gprob, -1.0), axis=1, keepdims=True)
    c1 = jnp.min(jnp.where((gprob == m1) & gmask, col, 999), axis=1,
                 keepdims=True)
    gp2 = jnp.where(col == c1, -1.0, jnp.where(gmask, gprob, -1.0))
    m2 = jnp.max(gp2, axis=1, keepdims=True)
    c2 = jnp.min(jnp.where((gp2 == m2) & gmask, col, 999), axis=1,
                 keepdims=True)

    elog = jnp.dot(xb, wr_ref[...].T, preferred_element_type=jnp.float32)
    elog = elog + brr_ref[...]
    r_i = lax.broadcasted_iota(jnp.int32, (128, 128), 0)
    c_i = lax.broadcasted_iota(jnp.int32, (128, 128), 1)
    pmat = (c_i == (r_i + 1 - 2 * (r_i % 2))).astype(jnp.float32)
    partner = jnp.dot(elog, pmat, preferred_element_type=jnp.float32)
    pm = jnp.maximum(elog, partner)
    ee = jnp.exp(elog - pm)
    ep = jnp.exp(partner - pm)
    eprob = ee / (ee + ep)

    h1 = jnp.dot(xb, wc1_ref[...].T, preferred_element_type=jnp.float32)
    h1 = _gelu_exact(h1 + bc1_ref[...])
    clog = jnp.sum(h1 * wc2_ref[...], axis=1, keepdims=True) + bc2_ref[...]
    conf = jax.nn.sigmoid(clog)

    grp = col // 2
    sel1 = grp == c1
    sel2 = grp == c2
    gp8 = jnp.where(sel1, m1, jnp.where(sel2, m2, 0.0))
    w8 = jnp.where(emask & (sel1 | sel2), eprob * gp8 * conf, 0.0)

    wm1 = jnp.max(w8, axis=1, keepdims=True)
    i1 = jnp.min(jnp.where((w8 == wm1) & emask, col, 999), axis=1,
                 keepdims=True)
    w8b = jnp.where(col == i1, -1.0, w8)
    wm2 = jnp.max(w8b, axis=1, keepdims=True)
    i2 = jnp.min(jnp.where((w8b == wm2) & emask, col, 999), axis=1,
                 keepdims=True)
    s = wm1 + wm2
    i1_ref[...] = i1
    i2_ref[...] = i2
    p1_ref[...] = wm1 / s
    p2_ref[...] = wm2 / s


def _route(xf, Wgp, bgr, Wrp, brr, Wc1, bc1r, Wc2, bc2r):
    full = lambda shape: pl.BlockSpec(shape, lambda b: (0,) * len(shape))
    return pl.pallas_call(
        _router_body,
        grid=(N // RBLK,),
        in_specs=[
            pl.BlockSpec((RBLK, D), lambda b: (b, 0)),
            full((128, D)), full((1, 128)),
            full((128, D)), full((1, 128)),
            full((D // 2, D)), full((1, D // 2)),
            full((1, D // 2)), full((1, 1)),
        ],
        out_specs=[
            pl.BlockSpec((RBLK, 1), lambda b: (b, 0)),
            pl.BlockSpec((RBLK, 1), lambda b: (b, 0)),
            pl.BlockSpec((RBLK, 1), lambda b: (b, 0)),
            pl.BlockSpec((RBLK, 1), lambda b: (b, 0)),
        ],
        out_shape=[
            jax.ShapeDtypeStruct((N, 1), jnp.int32),
            jax.ShapeDtypeStruct((N, 1), jnp.int32),
            jax.ShapeDtypeStruct((N, 1), jnp.float32),
            jax.ShapeDtypeStruct((N, 1), jnp.float32),
        ],
    )(xf, Wgp, bgr, Wrp, brr, Wc1, bc1r, Wc2, bc2r)


def _ffn_body(be_ref, xs_ref, ws_ref, w1_ref, b1_ref, w2_ref, b2_ref, zs_ref):
    xb = xs_ref[...]
    h = lax.dot_general(xb, w1_ref[0], (((1,), (1,)), ((), ())),
                        preferred_element_type=jnp.float32)
    h = _gelu_exact(h + b1_ref[0])
    y = lax.dot_general(h, w2_ref[0], (((1,), (1,)), ((), ())),
                        preferred_element_type=jnp.float32)
    y = y + b2_ref[0]
    zs_ref[...] = y * ws_ref[...]


def _ffn(xs, ws2, W1, b1, W2, b2, block_expert):
    grid_spec = pltpu.PrefetchScalarGridSpec(
        num_scalar_prefetch=1,
        grid=(NB,),
        in_specs=[
            pl.BlockSpec((BLK, D), lambda b, be: (b, 0)),
            pl.BlockSpec((BLK, 1), lambda b, be: (b, 0)),
            pl.BlockSpec((1, H, D), lambda b, be: (be[b], 0, 0)),
            pl.BlockSpec((1, 1, H), lambda b, be: (be[b], 0, 0)),
            pl.BlockSpec((1, D, H), lambda b, be: (be[b], 0, 0)),
            pl.BlockSpec((1, 1, D), lambda b, be: (be[b], 0, 0)),
        ],
        out_specs=pl.BlockSpec((BLK, D), lambda b, be: (b, 0)),
    )
    return pl.pallas_call(
        _ffn_body,
        grid_spec=grid_spec,
        out_shape=jax.ShapeDtypeStruct((P, D), jnp.float32),
    )(block_expert, xs, ws2, W1, b1, W2, b2)


def kernel(x, Wg, bg, Wr, br, Wc1, bc1, Wc2, bc2, W1, b1, W2, b2):
    xf = x.reshape(N, D)

    Wgp = jnp.zeros((128, D), jnp.float32).at[:G].set(Wg)
    bgr = jnp.zeros((1, 128), jnp.float32).at[0, :G].set(bg)
    Wrp = jnp.zeros((128, D), jnp.float32).at[:E].set(Wr.reshape(E, D))
    brr = jnp.zeros((1, 128), jnp.float32).at[0, :E].set(br.reshape(E))
    bc1r = bc1.reshape(1, D // 2)
    bc2r = bc2.reshape(1, 1)

    i1, i2, p1, p2 = _route(xf, Wgp, bgr, Wrp, brr, Wc1, bc1r, Wc2, bc2r)

    e_all = jnp.concatenate([i1[:, 0], i2[:, 0]])
    oh = (e_all[:, None] == jnp.arange(E)[None, :]).astype(jnp.int32)
    rank = jnp.take_along_axis(jnp.cumsum(oh, axis=0) - oh, e_all[:, None],
                               axis=1)[:, 0]
    cnt = oh.sum(0)
    padded = ((cnt + BLK - 1) // BLK) * BLK
    off = jnp.cumsum(padded) - padded
    pos = off[e_all] + rank
    tok = jnp.concatenate([jnp.arange(N), jnp.arange(N)])
    w_all = jnp.concatenate([p1[:, 0], p2[:, 0]])
    tok_sorted = jnp.zeros((P,), jnp.int32).at[pos].set(tok)
    ws = jnp.zeros((P,), jnp.float32).at[pos].set(w_all)
    bidx = jnp.arange(NB)
    block_expert = jnp.sum((bidx[:, None] * BLK >= off[None, :])
                           .astype(jnp.int32), axis=1) - 1

    xs = xf[tok_sorted]
    zs = _ffn(xs, ws[:, None], W1, b1[:, None, :], W2, b2[:, None, :],
              block_expert)
    out = zs[pos[:N]] + zs[pos[N:]]

    return out.reshape(x.shape), jnp.asarray(0.0, dtype=jnp.float32)
